# Initial kernel scaffold; baseline (speedup 1.0000x reference)
import jax
import jax.numpy as jnp
from jax.experimental import pallas as pl


def kernel(x, edge_index, W1l, b1, W1r, W2l, b2, W2r):
    raise NotImplementedError("write your pallas kernel here")


# speedup vs baseline: 5.7203x; 5.7203x over previous
import jax
import jax.numpy as jnp
from jax import lax
from jax.experimental import pallas as pl
from jax.experimental.pallas import tpu as pltpu
from jax.experimental.pallas import tpu_sc as plsc

N = 10000
E = 320000
D = 128

NC = 2
NS = 16
DH = D // NC
EPT = E // NS
CHUNK = 128
NCH = -(-EPT // CHUNK)
EPT_P = NCH * CHUNK
ACC_ROWS = 10240
ZRON = ACC_ROWS // NS // CHUNK
OUT_PER_TILE = ACC_ROWS // NS


def _sc_body(xs_hbm, srcp_hbm, dstp_hbm, out_hbm, deg_hbm,
             sidx_v, didx_v, rows_v, zbuf_v, ones_v, zero16_v, acc_sh, deg_sh):
    c = lax.axis_index("c")
    s = lax.axis_index("s")

    def fill_row(i, _):
        def fill_col(j, _):
            zbuf_v[i, pl.ds(j * 16, 16)] = jnp.zeros((16,), jnp.float32)
            return 0
        lax.fori_loop(0, DH // 16, fill_col, 0)
        ones_v[i] = jnp.ones((16,), jnp.float32)
        zero16_v[i] = jnp.zeros((16,), jnp.float32)
        return 0
    lax.fori_loop(0, CHUNK, fill_row, 0)

    zbase = s * OUT_PER_TILE
    for b in range(ZRON):
        pltpu.sync_copy(zbuf_v, acc_sh.at[pl.ds(zbase + b * CHUNK, CHUNK)])
        pltpu.sync_copy(zero16_v, deg_sh.at[pl.ds(zbase + b * CHUNK, CHUNK)])
    plsc.subcore_barrier()

    pltpu.sync_copy(srcp_hbm.at[s], sidx_v)
    pltpu.sync_copy(dstp_hbm.at[s], didx_v)

    def chunk_step(j, _):
        pltpu.sync_copy(xs_hbm.at[c].at[sidx_v.at[j]], rows_v)
        pltpu.sync_copy(rows_v, acc_sh.at[didx_v.at[j]], add=True)

        @pl.when(c == 0)
        def _():
            pltpu.sync_copy(ones_v, deg_sh.at[didx_v.at[j]], add=True)
        return 0
    lax.fori_loop(0, NCH, chunk_step, 0)
    plsc.subcore_barrier()

    obase = s * OUT_PER_TILE
    pltpu.sync_copy(acc_sh.at[pl.ds(obase, OUT_PER_TILE)],
                    out_hbm.at[c, pl.ds(obase, OUT_PER_TILE)])

    @pl.when(c == 0)
    def _():
        pltpu.sync_copy(deg_sh.at[pl.ds(obase, OUT_PER_TILE)],
                        deg_hbm.at[pl.ds(obase, OUT_PER_TILE)])


def _make_sc_pass():
    mesh = plsc.VectorSubcoreMesh(core_axis_name="c", subcore_axis_name="s",
                                  num_cores=NC, num_subcores=NS)
    out_type = [
        jax.ShapeDtypeStruct((NC, ACC_ROWS, DH), jnp.float32),
        jax.ShapeDtypeStruct((ACC_ROWS, 16), jnp.float32),
    ]
    scratch = [
        pltpu.VMEM((NCH, CHUNK), jnp.int32),
        pltpu.VMEM((NCH, CHUNK), jnp.int32),
        pltpu.VMEM((CHUNK, DH), jnp.float32),
        pltpu.VMEM((CHUNK, DH), jnp.float32),
        pltpu.VMEM((CHUNK, 16), jnp.float32),
        pltpu.VMEM((CHUNK, 16), jnp.float32),
        pltpu.VMEM_SHARED((ACC_ROWS, DH), jnp.float32),
        pltpu.VMEM_SHARED((ACC_ROWS, 16), jnp.float32),
    ]
    return pl.kernel(
        _sc_body,
        out_type=out_type,
        mesh=mesh,
        scratch_types=scratch,
        compiler_params=pltpu.CompilerParams(use_tc_tiling_on_sc=False),
    )


def _tc_body(p_ref, dg_ref, x_ref, wl_ref, b_ref, wr_ref, o_ref):
    psum = jnp.concatenate([p_ref[0], p_ref[1]], axis=1)
    deg = dg_ref[:, 0:1]
    agg = psum / jnp.maximum(deg, 1.0)
    z = lax.dot_general(agg, wl_ref[...], (((1,), (1,)), ((), ())),
                        preferred_element_type=jnp.float32)
    z = z + b_ref[...]
    z = z + lax.dot_general(x_ref[...], wr_ref[...], (((1,), (1,)), ((), ())),
                            preferred_element_type=jnp.float32)
    o_ref[...] = jnp.maximum(z, 0.01 * z)


BM = 1000


def _tc_pass(p, dg, x, wl, b, wr):
    grid = (N // BM,)
    return pl.pallas_call(
        _tc_body,
        grid=grid,
        in_specs=[
            pl.BlockSpec((NC, BM, DH), lambda i: (0, i, 0)),
            pl.BlockSpec((BM, 16), lambda i: (i, 0)),
            pl.BlockSpec((BM, D), lambda i: (i, 0)),
            pl.BlockSpec((D, D), lambda i: (0, 0)),
            pl.BlockSpec((1, D), lambda i: (0, 0)),
            pl.BlockSpec((D, D), lambda i: (0, 0)),
        ],
        out_specs=pl.BlockSpec((BM, D), lambda i: (i, 0)),
        out_shape=jax.ShapeDtypeStruct((N, D), jnp.float32),
    )(p, dg, x, wl, b.reshape(1, D), wr)


def kernel(x, edge_index, W1l, b1, W1r, W2l, b2, W2r):
    src = edge_index[0].reshape(NS, EPT)
    dst = edge_index[1].reshape(NS, EPT)
    pad = EPT_P - EPT
    srcp = jnp.pad(src, ((0, 0), (0, pad))).reshape(NS, NCH, CHUNK)
    dstp = jnp.pad(dst, ((0, 0), (0, pad)), constant_values=N).reshape(NS, NCH, CHUNK)

    sc = _make_sc_pass()

    xs = jnp.stack([x[:, :DH], x[:, DH:]])
    p1, dg = sc(xs, srcp, dstp)
    h1 = _tc_pass(p1, dg, x, W1l, b1, W1r)
    hs = jnp.stack([h1[:, :DH], h1[:, DH:]])
    p2, _ = sc(hs, srcp, dstp)
    h2 = _tc_pass(p2, dg, h1, W2l, b2, W2r)
    return h2


# speedup vs baseline: 6.2447x; 1.0917x over previous
import functools

import jax
import jax.numpy as jnp
from jax import lax
from jax.experimental import pallas as pl
from jax.experimental.pallas import tpu as pltpu
from jax.experimental.pallas import tpu_sc as plsc

N = 10000
E = 320000
D = 128

NC = 2
NS = 16
DH = D // NC
EPT = E // NS
CHUNK = 128
NBUF = 4
NCH = 160
EPT_P = NCH * CHUNK
ACC_ROWS = 10240
ZRON = ACC_ROWS // NS // CHUNK
OUT_PER_TILE = ACC_ROWS // NS


def _sc_body(with_deg, *refs):
    if with_deg:
        (xs_hbm, srcp_hbm, dstp_hbm, out_hbm, deg_hbm,
         sidx_v, didx_v, ones_v, zero16_v, acc_sh, deg_sh) = refs[:11]
        ring = refs[11:]
    else:
        (xs_hbm, srcp_hbm, dstp_hbm, out_hbm,
         sidx_v, didx_v, acc_sh) = refs[:7]
        ring = refs[7:]
    rows = ring[:NBUF]
    gs = ring[NBUF:2 * NBUF]
    ss = ring[2 * NBUF:3 * NBUF]

    c = lax.axis_index("c")
    s = lax.axis_index("s")

    def fill_row(i, _):
        def fill_col(j, _):
            rows[0][i, pl.ds(j * 16, 16)] = jnp.zeros((16,), jnp.float32)
            return 0
        lax.fori_loop(0, DH // 16, fill_col, 0)
        if with_deg:
            ones_v[i] = jnp.ones((16,), jnp.float32)
        return 0
    lax.fori_loop(0, CHUNK, fill_row, 0)

    if with_deg:
        def fill_z16(i, _):
            zero16_v[i] = jnp.zeros((16,), jnp.float32)
            return 0
        lax.fori_loop(0, 8, fill_z16, 0)

    pltpu.sync_copy(srcp_hbm.at[s], sidx_v)
    pltpu.sync_copy(dstp_hbm.at[s], didx_v)

    zbase = s * OUT_PER_TILE
    for b in range(ZRON):
        pltpu.sync_copy(rows[0], acc_sh.at[pl.ds(zbase + b * CHUNK, CHUNK)])
    if with_deg:
        def zero_deg(i, _):
            pltpu.sync_copy(zero16_v, deg_sh.at[pl.ds(zbase + i * 8, 8)])
            return 0
        lax.fori_loop(0, OUT_PER_TILE // 8, zero_deg, 0)
    plsc.subcore_barrier()

    def gather(j, b):
        return pltpu.async_copy(xs_hbm.at[c].at[sidx_v.at[j]], rows[b], gs[b])

    def gather_wait(j, b):
        pltpu.make_async_copy(xs_hbm.at[c].at[sidx_v.at[j]], rows[b],
                              gs[b]).wait()

    def scat_wait(j, b):
        pltpu.make_async_copy(rows[b], acc_sh.at[didx_v.at[j]], ss[b]).wait()

    for b in range(NBUF):
        gather(b, b)

    def group(g, _):
        for b in range(NBUF):
            j = NBUF * g + b
            gather_wait(j, b)
            pltpu.async_copy(rows[b], acc_sh.at[didx_v.at[j]], ss[b], add=True)
            if with_deg:
                @pl.when(c == b % 2)
                def _():
                    pltpu.sync_copy(ones_v, deg_sh.at[didx_v.at[j]], add=True)
            nb = (b + NBUF - 1) % NBUF

            @pl.when(j >= 1)
            def _():
                scat_wait(j - 1, nb)
                gather(j + NBUF - 1, nb)
        return 0
    lax.fori_loop(0, NCH // NBUF - 1, group, 0)

    scat_wait(NCH - NBUF - 1, NBUF - 1)
    gather(NCH - 1, NBUF - 1)
    for b in range(NBUF):
        j = NCH - NBUF + b
        gather_wait(j, b)
        pltpu.sync_copy(rows[b], acc_sh.at[didx_v.at[j]], add=True)
        if with_deg:
            @pl.when(c == b % 2)
            def _():
                pltpu.sync_copy(ones_v, deg_sh.at[didx_v.at[j]], add=True)
    plsc.subcore_barrier()

    obase = s * OUT_PER_TILE
    pltpu.sync_copy(acc_sh.at[pl.ds(obase, OUT_PER_TILE)],
                    out_hbm.at[c, pl.ds(obase, OUT_PER_TILE)])
    if with_deg:
        pltpu.sync_copy(deg_sh.at[pl.ds(obase, OUT_PER_TILE)],
                        deg_hbm.at[c, pl.ds(obase, OUT_PER_TILE)])


def _make_sc_pass(with_deg):
    mesh = plsc.VectorSubcoreMesh(core_axis_name="c", subcore_axis_name="s",
                                  num_cores=NC, num_subcores=NS)
    out_type = [jax.ShapeDtypeStruct((NC, ACC_ROWS, DH), jnp.float32)]
    scratch = [
        pltpu.VMEM((NCH, CHUNK), jnp.int32),
        pltpu.VMEM((NCH, CHUNK), jnp.int32),
    ]
    if with_deg:
        out_type.append(jax.ShapeDtypeStruct((NC, ACC_ROWS, 16), jnp.float32))
        scratch += [
            pltpu.VMEM((CHUNK, 16), jnp.float32),
            pltpu.VMEM((8, 16), jnp.float32),
        ]
    scratch.append(pltpu.VMEM_SHARED((ACC_ROWS, DH), jnp.float32))
    if with_deg:
        scratch.append(pltpu.VMEM_SHARED((ACC_ROWS, 16), jnp.float32))
    scratch += [pltpu.VMEM((CHUNK, DH), jnp.float32)] * NBUF \
        + [pltpu.SemaphoreType.DMA] * (2 * NBUF)
    return pl.kernel(
        functools.partial(_sc_body, with_deg),
        out_type=out_type,
        mesh=mesh,
        scratch_types=scratch,
        compiler_params=pltpu.CompilerParams(use_tc_tiling_on_sc=False),
    )


def _tc_body(p_ref, dg_ref, x_ref, wl_ref, b_ref, wr_ref, o_ref):
    psum = jnp.concatenate([p_ref[0], p_ref[1]], axis=1)
    deg = dg_ref[0, :, 0:1] + dg_ref[1, :, 0:1]
    agg = psum / jnp.maximum(deg, 1.0)
    z = lax.dot_general(agg, wl_ref[...], (((1,), (1,)), ((), ())),
                        preferred_element_type=jnp.float32)
    z = z + b_ref[...]
    z = z + lax.dot_general(x_ref[...], wr_ref[...], (((1,), (1,)), ((), ())),
                            preferred_element_type=jnp.float32)
    o_ref[...] = jnp.maximum(z, 0.01 * z)


BM = 1000


def _tc_pass(p, dg, x, wl, b, wr):
    grid = (N // BM,)
    return pl.pallas_call(
        _tc_body,
        grid=grid,
        in_specs=[
            pl.BlockSpec((NC, BM, DH), lambda i: (0, i, 0)),
            pl.BlockSpec((NC, BM, 16), lambda i: (0, i, 0)),
            pl.BlockSpec((BM, D), lambda i: (i, 0)),
            pl.BlockSpec((D, D), lambda i: (0, 0)),
            pl.BlockSpec((1, D), lambda i: (0, 0)),
            pl.BlockSpec((D, D), lambda i: (0, 0)),
        ],
        out_specs=pl.BlockSpec((BM, D), lambda i: (i, 0)),
        out_shape=jax.ShapeDtypeStruct((N, D), jnp.float32),
    )(p, dg, x, wl, b.reshape(1, D), wr)


def kernel(x, edge_index, W1l, b1, W1r, W2l, b2, W2r):
    src = edge_index[0].reshape(NS, EPT)
    dst = edge_index[1].reshape(NS, EPT)
    pad = EPT_P - EPT
    srcp = jnp.pad(src, ((0, 0), (0, pad))).reshape(NS, NCH, CHUNK)
    dstp = jnp.pad(dst, ((0, 0), (0, pad)), constant_values=N).reshape(NS, NCH, CHUNK)

    sc_deg = _make_sc_pass(True)
    sc_plain = _make_sc_pass(False)

    xs = jnp.stack([x[:, :DH], x[:, DH:]])
    p1, dg = sc_deg(xs, srcp, dstp)
    h1 = _tc_pass(p1, dg, x, W1l, b1, W1r)
    hs = jnp.stack([h1[:, :DH], h1[:, DH:]])
    (p2,) = sc_plain(hs, srcp, dstp)
    h2 = _tc_pass(p2, dg, h1, W2l, b2, W2r)
    return h2


# speedup vs baseline: 6.2712x; 1.0042x over previous
import functools

import jax
import jax.numpy as jnp
from jax import lax
from jax.experimental import pallas as pl
from jax.experimental.pallas import tpu as pltpu
from jax.experimental.pallas import tpu_sc as plsc

N = 10000
E = 320000
D = 128

NC = 2
NS = 16
DH = D // NC
EPT = E // NS
CHUNK = 128
NBUF = 4
NCH = 160
EPT_P = NCH * CHUNK
ACC_ROWS = 10240
ZRON = ACC_ROWS // NS // CHUNK
OUT_PER_TILE = ACC_ROWS // NS


def _sc_body(with_deg, *refs):
    if with_deg:
        (xs_hbm, srcp_hbm, dstp_hbm, out_hbm, deg_hbm,
         sidx_v, didx_v, ones_v, zero16_v, acc_sh, deg_sh) = refs[:11]
        ring = refs[11:]
    else:
        (xs_hbm, srcp_hbm, dstp_hbm, out_hbm,
         sidx_v, didx_v, acc_sh) = refs[:7]
        ring = refs[7:]
    rows = ring[:NBUF]
    gs = ring[NBUF:2 * NBUF]
    ss = ring[2 * NBUF:3 * NBUF]

    c = lax.axis_index("c")
    s = lax.axis_index("s")

    def fill_row(i, _):
        def fill_col(j, _):
            rows[0][i, pl.ds(j * 16, 16)] = jnp.zeros((16,), jnp.float32)
            return 0
        lax.fori_loop(0, DH // 16, fill_col, 0)
        if with_deg:
            ones_v[i] = jnp.ones((16,), jnp.float32)
        return 0
    lax.fori_loop(0, CHUNK, fill_row, 0)

    if with_deg:
        def fill_z16(i, _):
            zero16_v[i] = jnp.zeros((16,), jnp.float32)
            return 0
        lax.fori_loop(0, CHUNK, fill_z16, 0)

    pltpu.sync_copy(srcp_hbm.at[s], sidx_v)
    pltpu.sync_copy(dstp_hbm.at[s], didx_v)

    zbase = s * OUT_PER_TILE
    for b in range(ZRON):
        pltpu.sync_copy(rows[0], acc_sh.at[pl.ds(zbase + b * CHUNK, CHUNK)])
    if with_deg:
        for b in range(ZRON):
            pltpu.sync_copy(zero16_v,
                            deg_sh.at[pl.ds(zbase + b * CHUNK, CHUNK)])
    plsc.subcore_barrier()

    def gather(j, b):
        return pltpu.async_copy(xs_hbm.at[c].at[sidx_v.at[j]], rows[b], gs[b])

    def gather_wait(j, b):
        pltpu.make_async_copy(xs_hbm.at[c].at[sidx_v.at[j]], rows[b],
                              gs[b]).wait()

    def scat_wait(j, b):
        pltpu.make_async_copy(rows[b], acc_sh.at[didx_v.at[j]], ss[b]).wait()

    for b in range(NBUF):
        gather(b, b)

    def group(g, _):
        for b in range(NBUF):
            j = NBUF * g + b
            gather_wait(j, b)
            pltpu.async_copy(rows[b], acc_sh.at[didx_v.at[j]], ss[b], add=True)
            if with_deg:
                @pl.when(c == b % 2)
                def _():
                    pltpu.sync_copy(ones_v, deg_sh.at[didx_v.at[j]], add=True)
            nb = (b + NBUF - 1) % NBUF

            @pl.when(j >= 1)
            def _():
                scat_wait(j - 1, nb)
                gather(j + NBUF - 1, nb)
        return 0
    lax.fori_loop(0, NCH // NBUF - 1, group, 0)

    scat_wait(NCH - NBUF - 1, NBUF - 1)
    gather(NCH - 1, NBUF - 1)
    for b in range(NBUF):
        j = NCH - NBUF + b
        gather_wait(j, b)
        pltpu.sync_copy(rows[b], acc_sh.at[didx_v.at[j]], add=True)
        if with_deg:
            @pl.when(c == b % 2)
            def _():
                pltpu.sync_copy(ones_v, deg_sh.at[didx_v.at[j]], add=True)
    plsc.subcore_barrier()

    obase = s * OUT_PER_TILE
    pltpu.sync_copy(acc_sh.at[pl.ds(obase, OUT_PER_TILE)],
                    out_hbm.at[c, pl.ds(obase, OUT_PER_TILE)])
    if with_deg:
        pltpu.sync_copy(deg_sh.at[pl.ds(obase, OUT_PER_TILE)],
                        deg_hbm.at[c, pl.ds(obase, OUT_PER_TILE)])


def _make_sc_pass(with_deg):
    mesh = plsc.VectorSubcoreMesh(core_axis_name="c", subcore_axis_name="s",
                                  num_cores=NC, num_subcores=NS)
    out_type = [jax.ShapeDtypeStruct((NC, ACC_ROWS, DH), jnp.float32)]
    scratch = [
        pltpu.VMEM((NCH, CHUNK), jnp.int32),
        pltpu.VMEM((NCH, CHUNK), jnp.int32),
    ]
    if with_deg:
        out_type.append(jax.ShapeDtypeStruct((NC, ACC_ROWS, 16), jnp.float32))
        scratch += [
            pltpu.VMEM((CHUNK, 16), jnp.float32),
            pltpu.VMEM((CHUNK, 16), jnp.float32),
        ]
    scratch.append(pltpu.VMEM_SHARED((ACC_ROWS, DH), jnp.float32))
    if with_deg:
        scratch.append(pltpu.VMEM_SHARED((ACC_ROWS, 16), jnp.float32))
    scratch += [pltpu.VMEM((CHUNK, DH), jnp.float32)] * NBUF \
        + [pltpu.SemaphoreType.DMA] * (2 * NBUF)
    return pl.kernel(
        functools.partial(_sc_body, with_deg),
        out_type=out_type,
        mesh=mesh,
        scratch_types=scratch,
        compiler_params=pltpu.CompilerParams(use_tc_tiling_on_sc=False),
    )


def _tc_body(p_ref, dg_ref, x_ref, wl_ref, b_ref, wr_ref, o_ref):
    psum = jnp.concatenate([p_ref[0], p_ref[1]], axis=1)
    deg = dg_ref[0, :, 0:1] + dg_ref[1, :, 0:1]
    agg = psum / jnp.maximum(deg, 1.0)
    z = lax.dot_general(agg, wl_ref[...], (((1,), (1,)), ((), ())),
                        preferred_element_type=jnp.float32)
    z = z + b_ref[...]
    z = z + lax.dot_general(x_ref[...], wr_ref[...], (((1,), (1,)), ((), ())),
                            preferred_element_type=jnp.float32)
    o_ref[...] = jnp.maximum(z, 0.01 * z)


BM = 1000


def _tc_pass(p, dg, x, wl, b, wr):
    grid = (N // BM,)
    return pl.pallas_call(
        _tc_body,
        grid=grid,
        in_specs=[
            pl.BlockSpec((NC, BM, DH), lambda i: (0, i, 0)),
            pl.BlockSpec((NC, BM, 16), lambda i: (0, i, 0)),
            pl.BlockSpec((BM, D), lambda i: (i, 0)),
            pl.BlockSpec((D, D), lambda i: (0, 0)),
            pl.BlockSpec((1, D), lambda i: (0, 0)),
            pl.BlockSpec((D, D), lambda i: (0, 0)),
        ],
        out_specs=pl.BlockSpec((BM, D), lambda i: (i, 0)),
        out_shape=jax.ShapeDtypeStruct((N, D), jnp.float32),
    )(p, dg, x, wl, b.reshape(1, D), wr)


def kernel(x, edge_index, W1l, b1, W1r, W2l, b2, W2r):
    src = edge_index[0].reshape(NS, EPT)
    dst = edge_index[1].reshape(NS, EPT)
    pad = EPT_P - EPT
    srcp = jnp.pad(src, ((0, 0), (0, pad))).reshape(NS, NCH, CHUNK)
    dstp = jnp.pad(dst, ((0, 0), (0, pad)), constant_values=N).reshape(NS, NCH, CHUNK)

    sc_deg = _make_sc_pass(True)
    sc_plain = _make_sc_pass(False)

    xs = jnp.stack([x[:, :DH], x[:, DH:]])
    p1, dg = sc_deg(xs, srcp, dstp)
    h1 = _tc_pass(p1, dg, x, W1l, b1, W1r)
    hs = jnp.stack([h1[:, :DH], h1[:, DH:]])
    (p2,) = sc_plain(hs, srcp, dstp)
    h2 = _tc_pass(p2, dg, h1, W2l, b2, W2r)
    return h2


# speedup vs baseline: 6.3223x; 1.0081x over previous
import functools

import jax
import jax.numpy as jnp
from jax import lax
from jax.experimental import pallas as pl
from jax.experimental.pallas import tpu as pltpu
from jax.experimental.pallas import tpu_sc as plsc

N = 10000
E = 320000
D = 128

NC = 2
NS = 16
DH = D // NC
EPT = E // NS
CHUNK = 128
NBUF = 4
NCH = 160
EPT_P = NCH * CHUNK
ACC_ROWS = 10240
ZRON = ACC_ROWS // NS // CHUNK
OUT_PER_TILE = ACC_ROWS // NS


def _sc_body(with_deg, *refs):
    if with_deg:
        (xs_hbm, srcp_hbm, dstp_hbm, out_hbm, deg_hbm,
         sidx_v, didx_v, ones_v, zero16_v, acc_sh, deg_sh) = refs[:11]
        ring = refs[11:]
    else:
        (xs_hbm, srcp_hbm, dstp_hbm, out_hbm,
         sidx_v, didx_v, acc_sh) = refs[:7]
        ring = refs[7:]
    rows = ring[:NBUF]
    gs = ring[NBUF:2 * NBUF]
    ss = ring[2 * NBUF:3 * NBUF]

    c = lax.axis_index("c")
    s = lax.axis_index("s")

    def fill_row(i, _):
        def fill_col(j, _):
            rows[0][i, pl.ds(j * 16, 16)] = jnp.zeros((16,), jnp.float32)
            return 0
        lax.fori_loop(0, DH // 16, fill_col, 0)
        if with_deg:
            ones_v[i] = jnp.ones((16,), jnp.float32)
        return 0
    lax.fori_loop(0, CHUNK, fill_row, 0)

    if with_deg:
        def fill_z16(i, _):
            zero16_v[i] = jnp.zeros((16,), jnp.float32)
            return 0
        lax.fori_loop(0, CHUNK, fill_z16, 0)

    pltpu.sync_copy(srcp_hbm.at[s], sidx_v)
    pltpu.sync_copy(dstp_hbm.at[s], didx_v)

    zbase = s * OUT_PER_TILE
    for b in range(ZRON):
        pltpu.sync_copy(rows[0], acc_sh.at[pl.ds(zbase + b * CHUNK, CHUNK)])
    if with_deg:
        for b in range(ZRON):
            pltpu.sync_copy(zero16_v,
                            deg_sh.at[pl.ds(zbase + b * CHUNK, CHUNK)])
    plsc.subcore_barrier()

    def gather(j, b):
        return pltpu.async_copy(xs_hbm.at[c].at[sidx_v.at[j]], rows[b], gs[b])

    def gather_wait(j, b):
        pltpu.make_async_copy(xs_hbm.at[c].at[sidx_v.at[j]], rows[b],
                              gs[b]).wait()

    def scat_wait(j, b):
        pltpu.make_async_copy(rows[b], acc_sh.at[didx_v.at[j]], ss[b]).wait()

    for b in range(NBUF):
        gather(b, b)

    def group(g, _):
        for b in range(NBUF):
            j = NBUF * g + b
            gather_wait(j, b)
            pltpu.async_copy(rows[b], acc_sh.at[didx_v.at[j]], ss[b], add=True)
            if with_deg:
                @pl.when(c == b % 2)
                def _():
                    pltpu.sync_copy(ones_v, deg_sh.at[didx_v.at[j]], add=True)
            nb = (b + NBUF - 1) % NBUF

            @pl.when(j >= 1)
            def _():
                scat_wait(j - 1, nb)
                gather(j + NBUF - 1, nb)
        return 0
    lax.fori_loop(0, NCH // NBUF - 1, group, 0)

    scat_wait(NCH - NBUF - 1, NBUF - 1)
    gather(NCH - 1, NBUF - 1)
    for b in range(NBUF):
        j = NCH - NBUF + b
        gather_wait(j, b)
        pltpu.sync_copy(rows[b], acc_sh.at[didx_v.at[j]], add=True)
        if with_deg:
            @pl.when(c == b % 2)
            def _():
                pltpu.sync_copy(ones_v, deg_sh.at[didx_v.at[j]], add=True)
    plsc.subcore_barrier()

    obase = s * OUT_PER_TILE
    pltpu.sync_copy(acc_sh.at[pl.ds(obase, OUT_PER_TILE)],
                    out_hbm.at[c, pl.ds(obase, OUT_PER_TILE)])
    if with_deg:
        pltpu.sync_copy(deg_sh.at[pl.ds(obase, OUT_PER_TILE)],
                        deg_hbm.at[c, pl.ds(obase, OUT_PER_TILE)])


def _make_sc_pass(with_deg):
    mesh = plsc.VectorSubcoreMesh(core_axis_name="c", subcore_axis_name="s",
                                  num_cores=NC, num_subcores=NS)
    out_type = [jax.ShapeDtypeStruct((NC, ACC_ROWS, DH), jnp.float32)]
    scratch = [
        pltpu.VMEM((NCH, CHUNK), jnp.int32),
        pltpu.VMEM((NCH, CHUNK), jnp.int32),
    ]
    if with_deg:
        out_type.append(jax.ShapeDtypeStruct((NC, ACC_ROWS, 16), jnp.float32))
        scratch += [
            pltpu.VMEM((CHUNK, 16), jnp.float32),
            pltpu.VMEM((CHUNK, 16), jnp.float32),
        ]
    scratch.append(pltpu.VMEM_SHARED((ACC_ROWS, DH), jnp.float32))
    if with_deg:
        scratch.append(pltpu.VMEM_SHARED((ACC_ROWS, 16), jnp.float32))
    scratch += [pltpu.VMEM((CHUNK, DH), jnp.float32)] * NBUF \
        + [pltpu.SemaphoreType.DMA] * (2 * NBUF)
    return pl.kernel(
        functools.partial(_sc_body, with_deg),
        out_type=out_type,
        mesh=mesh,
        scratch_types=scratch,
        compiler_params=pltpu.CompilerParams(use_tc_tiling_on_sc=False),
    )


def _tc_body(p_ref, dg_ref, x_ref, wl_ref, b_ref, wr_ref, o_ref):
    psum = jnp.concatenate([p_ref[0], p_ref[1]], axis=1)
    deg = dg_ref[0, :, 0:1] + dg_ref[1, :, 0:1]
    agg = psum / jnp.maximum(deg, 1.0)
    z = lax.dot_general(agg, wl_ref[...], (((1,), (1,)), ((), ())),
                        preferred_element_type=jnp.float32)
    z = z + b_ref[...]
    z = z + lax.dot_general(x_ref[...], wr_ref[...], (((1,), (1,)), ((), ())),
                            preferred_element_type=jnp.float32)
    o_ref[...] = jnp.maximum(z, 0.01 * z)


BM = 2000


def _tc_pass(p, dg, x, wl, b, wr):
    grid = (N // BM,)
    return pl.pallas_call(
        _tc_body,
        grid=grid,
        in_specs=[
            pl.BlockSpec((NC, BM, DH), lambda i: (0, i, 0)),
            pl.BlockSpec((NC, BM, 16), lambda i: (0, i, 0)),
            pl.BlockSpec((BM, D), lambda i: (i, 0)),
            pl.BlockSpec((D, D), lambda i: (0, 0)),
            pl.BlockSpec((1, D), lambda i: (0, 0)),
            pl.BlockSpec((D, D), lambda i: (0, 0)),
        ],
        out_specs=pl.BlockSpec((BM, D), lambda i: (i, 0)),
        out_shape=jax.ShapeDtypeStruct((N, D), jnp.float32),
    )(p, dg, x, wl, b.reshape(1, D), wr)


def kernel(x, edge_index, W1l, b1, W1r, W2l, b2, W2r):
    src = edge_index[0].reshape(NS, EPT)
    dst = edge_index[1].reshape(NS, EPT)
    pad = EPT_P - EPT
    srcp = jnp.pad(src, ((0, 0), (0, pad))).reshape(NS, NCH, CHUNK)
    dstp = jnp.pad(dst, ((0, 0), (0, pad)), constant_values=N).reshape(NS, NCH, CHUNK)

    sc_deg = _make_sc_pass(True)
    sc_plain = _make_sc_pass(False)

    xs = jnp.stack([x[:, :DH], x[:, DH:]])
    p1, dg = sc_deg(xs, srcp, dstp)
    h1 = _tc_pass(p1, dg, x, W1l, b1, W1r)
    hs = jnp.stack([h1[:, :DH], h1[:, DH:]])
    (p2,) = sc_plain(hs, srcp, dstp)
    h2 = _tc_pass(p2, dg, h1, W2l, b2, W2r)
    return h2


# speedup vs baseline: 6.3294x; 1.0011x over previous
import functools

import jax
import jax.numpy as jnp
from jax import lax
from jax.experimental import pallas as pl
from jax.experimental.pallas import tpu as pltpu
from jax.experimental.pallas import tpu_sc as plsc

N = 10000
E = 320000
D = 128

NC = 2
NS = 16
DH = D // NC
EPT = E // NS
CHUNK = 128
NBUF = 4
NCH = 160
EPT_P = NCH * CHUNK
ACC_ROWS = 10240
ZRON = ACC_ROWS // NS // CHUNK
OUT_PER_TILE = ACC_ROWS // NS


def _sc_body(with_deg, *refs):
    if with_deg:
        (xs_hbm, srcp_hbm, dstp_hbm, out_hbm, deg_hbm,
         sidx_v, didx_v, ones_v, zero16_v, acc_sh, deg_sh) = refs[:11]
        ring = refs[11:]
    else:
        (xs_hbm, srcp_hbm, dstp_hbm, out_hbm,
         sidx_v, didx_v, acc_sh) = refs[:7]
        ring = refs[7:]
    rows = ring[:NBUF]
    gs = ring[NBUF:2 * NBUF]
    ss = ring[2 * NBUF:3 * NBUF]

    c = lax.axis_index("c")
    s = lax.axis_index("s")

    def fill_row(i, _):
        def fill_col(j, _):
            rows[0][i, pl.ds(j * 16, 16)] = jnp.zeros((16,), jnp.float32)
            return 0
        lax.fori_loop(0, DH // 16, fill_col, 0)
        if with_deg:
            ones_v[i] = jnp.ones((16,), jnp.float32)
        return 0
    lax.fori_loop(0, CHUNK, fill_row, 0)

    if with_deg:
        def fill_z16(i, _):
            zero16_v[i] = jnp.zeros((16,), jnp.float32)
            return 0
        lax.fori_loop(0, CHUNK, fill_z16, 0)

    pltpu.sync_copy(srcp_hbm.at[s], sidx_v)
    pltpu.sync_copy(dstp_hbm.at[s], didx_v)

    zbase = s * OUT_PER_TILE
    for b in range(ZRON):
        pltpu.sync_copy(rows[0], acc_sh.at[pl.ds(zbase + b * CHUNK, CHUNK)])
    if with_deg:
        for b in range(ZRON):
            pltpu.sync_copy(zero16_v,
                            deg_sh.at[pl.ds(zbase + b * CHUNK, CHUNK)])
    plsc.subcore_barrier()

    def gather(j, b):
        return pltpu.async_copy(xs_hbm.at[c].at[sidx_v.at[j]], rows[b], gs[b])

    def gather_wait(j, b):
        pltpu.make_async_copy(xs_hbm.at[c].at[sidx_v.at[j]], rows[b],
                              gs[b]).wait()

    def scat_wait(j, b):
        pltpu.make_async_copy(rows[b], acc_sh.at[didx_v.at[j]], ss[b]).wait()

    for b in range(NBUF):
        gather(b, b)

    def group(g, _):
        for b in range(NBUF):
            j = NBUF * g + b
            gather_wait(j, b)
            pltpu.async_copy(rows[b], acc_sh.at[didx_v.at[j]], ss[b], add=True)
            if with_deg:
                @pl.when(c == b % 2)
                def _():
                    pltpu.sync_copy(ones_v, deg_sh.at[didx_v.at[j]], add=True)
            nb = (b + NBUF - 1) % NBUF

            @pl.when(j >= 1)
            def _():
                scat_wait(j - 1, nb)
                gather(j + NBUF - 1, nb)
        return 0
    lax.fori_loop(0, NCH // NBUF - 1, group, 0)

    scat_wait(NCH - NBUF - 1, NBUF - 1)
    gather(NCH - 1, NBUF - 1)
    for b in range(NBUF):
        j = NCH - NBUF + b
        gather_wait(j, b)
        pltpu.sync_copy(rows[b], acc_sh.at[didx_v.at[j]], add=True)
        if with_deg:
            @pl.when(c == b % 2)
            def _():
                pltpu.sync_copy(ones_v, deg_sh.at[didx_v.at[j]], add=True)
    plsc.subcore_barrier()

    obase = s * OUT_PER_TILE
    pltpu.sync_copy(acc_sh.at[pl.ds(obase, OUT_PER_TILE)],
                    out_hbm.at[c, pl.ds(obase, OUT_PER_TILE)])
    if with_deg:
        pltpu.sync_copy(deg_sh.at[pl.ds(obase, OUT_PER_TILE)],
                        deg_hbm.at[c, pl.ds(obase, OUT_PER_TILE)])


def _make_sc_pass(with_deg):
    mesh = plsc.VectorSubcoreMesh(core_axis_name="c", subcore_axis_name="s",
                                  num_cores=NC, num_subcores=NS)
    out_type = [jax.ShapeDtypeStruct((NC, ACC_ROWS, DH), jnp.float32)]
    scratch = [
        pltpu.VMEM((NCH, CHUNK), jnp.int32),
        pltpu.VMEM((NCH, CHUNK), jnp.int32),
    ]
    if with_deg:
        out_type.append(jax.ShapeDtypeStruct((NC, ACC_ROWS, 16), jnp.float32))
        scratch += [
            pltpu.VMEM((CHUNK, 16), jnp.float32),
            pltpu.VMEM((CHUNK, 16), jnp.float32),
        ]
    scratch.append(pltpu.VMEM_SHARED((ACC_ROWS, DH), jnp.float32))
    if with_deg:
        scratch.append(pltpu.VMEM_SHARED((ACC_ROWS, 16), jnp.float32))
    scratch += [pltpu.VMEM((CHUNK, DH), jnp.float32)] * NBUF \
        + [pltpu.SemaphoreType.DMA] * (2 * NBUF)
    return pl.kernel(
        functools.partial(_sc_body, with_deg),
        out_type=out_type,
        mesh=mesh,
        scratch_types=scratch,
        compiler_params=pltpu.CompilerParams(use_tc_tiling_on_sc=False),
    )


def _tc_body(p_ref, dg_ref, x_ref, wl_ref, b_ref, wr_ref, o_ref):
    psum = jnp.concatenate([p_ref[0], p_ref[1]], axis=1)
    deg = dg_ref[0, :, 0:1] + dg_ref[1, :, 0:1]
    agg = psum / jnp.maximum(deg, 1.0)
    z = lax.dot_general(agg, wl_ref[...], (((1,), (1,)), ((), ())),
                        preferred_element_type=jnp.float32)
    z = z + b_ref[...]
    z = z + lax.dot_general(x_ref[...], wr_ref[...], (((1,), (1,)), ((), ())),
                            preferred_element_type=jnp.float32)
    o_ref[...] = jnp.maximum(z, 0.01 * z)


BM = 5000


def _tc_pass(p, dg, x, wl, b, wr):
    grid = (N // BM,)
    return pl.pallas_call(
        _tc_body,
        grid=grid,
        in_specs=[
            pl.BlockSpec((NC, BM, DH), lambda i: (0, i, 0)),
            pl.BlockSpec((NC, BM, 16), lambda i: (0, i, 0)),
            pl.BlockSpec((BM, D), lambda i: (i, 0)),
            pl.BlockSpec((D, D), lambda i: (0, 0)),
            pl.BlockSpec((1, D), lambda i: (0, 0)),
            pl.BlockSpec((D, D), lambda i: (0, 0)),
        ],
        out_specs=pl.BlockSpec((BM, D), lambda i: (i, 0)),
        out_shape=jax.ShapeDtypeStruct((N, D), jnp.float32),
    )(p, dg, x, wl, b.reshape(1, D), wr)


def kernel(x, edge_index, W1l, b1, W1r, W2l, b2, W2r):
    src = edge_index[0].reshape(NS, EPT)
    dst = edge_index[1].reshape(NS, EPT)
    pad = EPT_P - EPT
    srcp = jnp.pad(src, ((0, 0), (0, pad))).reshape(NS, NCH, CHUNK)
    dstp = jnp.pad(dst, ((0, 0), (0, pad)), constant_values=N).reshape(NS, NCH, CHUNK)

    sc_deg = _make_sc_pass(True)
    sc_plain = _make_sc_pass(False)

    xs = jnp.stack([x[:, :DH], x[:, DH:]])
    p1, dg = sc_deg(xs, srcp, dstp)
    h1 = _tc_pass(p1, dg, x, W1l, b1, W1r)
    hs = jnp.stack([h1[:, :DH], h1[:, DH:]])
    (p2,) = sc_plain(hs, srcp, dstp)
    h2 = _tc_pass(p2, dg, h1, W2l, b2, W2r)
    return h2


# speedup vs baseline: 6.3461x; 1.0026x over previous
import functools

import jax
import jax.numpy as jnp
from jax import lax
from jax.experimental import pallas as pl
from jax.experimental.pallas import tpu as pltpu
from jax.experimental.pallas import tpu_sc as plsc

N = 10000
E = 320000
D = 128

NC = 2
NS = 16
DH = D // NC
EPT = E // NS
CHUNK = 128
NBUF = 4
NCH = 160
EPT_P = NCH * CHUNK
ACC_ROWS = 10240
ZRON = ACC_ROWS // NS // CHUNK
OUT_PER_TILE = ACC_ROWS // NS


def _sc_body(with_deg, *refs):
    if with_deg:
        (xs_hbm, srcp_hbm, dstp_hbm, out_hbm, deg_hbm,
         sidx_v, didx_v, ones_v, zero16_v, acc_sh, deg_sh) = refs[:11]
        ring = refs[11:]
    else:
        (xs_hbm, srcp_hbm, dstp_hbm, out_hbm,
         sidx_v, didx_v, acc_sh) = refs[:7]
        ring = refs[7:]
    rows = ring[:NBUF]
    gs = ring[NBUF:2 * NBUF]
    ss = ring[2 * NBUF:3 * NBUF]

    c = lax.axis_index("c")
    s = lax.axis_index("s")

    def fill_row(i, _):
        def fill_col(j, _):
            rows[0][i, pl.ds(j * 16, 16)] = jnp.zeros((16,), jnp.float32)
            return 0
        lax.fori_loop(0, DH // 16, fill_col, 0)
        if with_deg:
            ones_v[i] = jnp.ones((16,), jnp.float32)
        return 0
    lax.fori_loop(0, CHUNK, fill_row, 0)

    if with_deg:
        def fill_z16(i, _):
            zero16_v[i] = jnp.zeros((16,), jnp.float32)
            return 0
        lax.fori_loop(0, CHUNK, fill_z16, 0)

    pltpu.sync_copy(srcp_hbm.at[s], sidx_v)
    pltpu.sync_copy(dstp_hbm.at[s], didx_v)

    def gather(j, b):
        return pltpu.async_copy(xs_hbm.at[c].at[sidx_v.at[j]], rows[b], gs[b])

    def gather_wait(j, b):
        pltpu.make_async_copy(xs_hbm.at[c].at[sidx_v.at[j]], rows[b],
                              gs[b]).wait()

    def scat_wait(j, b):
        pltpu.make_async_copy(rows[b], acc_sh.at[didx_v.at[j]], ss[b]).wait()

    for b in range(1, NBUF):
        gather(b, b)

    zbase = s * OUT_PER_TILE
    for b in range(ZRON):
        pltpu.sync_copy(rows[0], acc_sh.at[pl.ds(zbase + b * CHUNK, CHUNK)])
    if with_deg:
        for b in range(ZRON):
            pltpu.sync_copy(zero16_v,
                            deg_sh.at[pl.ds(zbase + b * CHUNK, CHUNK)])
    gather(0, 0)
    plsc.subcore_barrier()

    def group(g, _):
        for b in range(NBUF):
            j = NBUF * g + b
            gather_wait(j, b)
            pltpu.async_copy(rows[b], acc_sh.at[didx_v.at[j]], ss[b], add=True)
            if with_deg:
                @pl.when(c == b % 2)
                def _():
                    pltpu.sync_copy(ones_v, deg_sh.at[didx_v.at[j]], add=True)
            nb = (b + NBUF - 1) % NBUF

            @pl.when(j >= 1)
            def _():
                scat_wait(j - 1, nb)
                gather(j + NBUF - 1, nb)
        return 0
    lax.fori_loop(0, NCH // NBUF - 1, group, 0)

    scat_wait(NCH - NBUF - 1, NBUF - 1)
    gather(NCH - 1, NBUF - 1)
    for b in range(NBUF):
        j = NCH - NBUF + b
        gather_wait(j, b)
        pltpu.sync_copy(rows[b], acc_sh.at[didx_v.at[j]], add=True)
        if with_deg:
            @pl.when(c == b % 2)
            def _():
                pltpu.sync_copy(ones_v, deg_sh.at[didx_v.at[j]], add=True)
    plsc.subcore_barrier()

    obase = s * OUT_PER_TILE
    pltpu.sync_copy(acc_sh.at[pl.ds(obase, OUT_PER_TILE)],
                    out_hbm.at[c, pl.ds(obase, OUT_PER_TILE)])
    if with_deg:
        pltpu.sync_copy(deg_sh.at[pl.ds(obase, OUT_PER_TILE)],
                        deg_hbm.at[c, pl.ds(obase, OUT_PER_TILE)])


def _make_sc_pass(with_deg):
    mesh = plsc.VectorSubcoreMesh(core_axis_name="c", subcore_axis_name="s",
                                  num_cores=NC, num_subcores=NS)
    out_type = [jax.ShapeDtypeStruct((NC, ACC_ROWS, DH), jnp.float32)]
    scratch = [
        pltpu.VMEM((NCH, CHUNK), jnp.int32),
        pltpu.VMEM((NCH, CHUNK), jnp.int32),
    ]
    if with_deg:
        out_type.append(jax.ShapeDtypeStruct((NC, ACC_ROWS, 16), jnp.float32))
        scratch += [
            pltpu.VMEM((CHUNK, 16), jnp.float32),
            pltpu.VMEM((CHUNK, 16), jnp.float32),
        ]
    scratch.append(pltpu.VMEM_SHARED((ACC_ROWS, DH), jnp.float32))
    if with_deg:
        scratch.append(pltpu.VMEM_SHARED((ACC_ROWS, 16), jnp.float32))
    scratch += [pltpu.VMEM((CHUNK, DH), jnp.float32)] * NBUF \
        + [pltpu.SemaphoreType.DMA] * (2 * NBUF)
    return pl.kernel(
        functools.partial(_sc_body, with_deg),
        out_type=out_type,
        mesh=mesh,
        scratch_types=scratch,
        compiler_params=pltpu.CompilerParams(use_tc_tiling_on_sc=False),
    )


def _tc_body(p_ref, dg_ref, x_ref, wl_ref, b_ref, wr_ref, o_ref):
    psum = jnp.concatenate([p_ref[0], p_ref[1]], axis=1)
    deg = dg_ref[0, :, 0:1] + dg_ref[1, :, 0:1]
    agg = psum / jnp.maximum(deg, 1.0)
    z = lax.dot_general(agg, wl_ref[...], (((1,), (1,)), ((), ())),
                        preferred_element_type=jnp.float32)
    z = z + b_ref[...]
    z = z + lax.dot_general(x_ref[...], wr_ref[...], (((1,), (1,)), ((), ())),
                            preferred_element_type=jnp.float32)
    o_ref[...] = jnp.maximum(z, 0.01 * z)


BM = 5000


def _tc_pass(p, dg, x, wl, b, wr):
    grid = (N // BM,)
    return pl.pallas_call(
        _tc_body,
        grid=grid,
        in_specs=[
            pl.BlockSpec((NC, BM, DH), lambda i: (0, i, 0)),
            pl.BlockSpec((NC, BM, 16), lambda i: (0, i, 0)),
            pl.BlockSpec((BM, D), lambda i: (i, 0)),
            pl.BlockSpec((D, D), lambda i: (0, 0)),
            pl.BlockSpec((1, D), lambda i: (0, 0)),
            pl.BlockSpec((D, D), lambda i: (0, 0)),
        ],
        out_specs=pl.BlockSpec((BM, D), lambda i: (i, 0)),
        out_shape=jax.ShapeDtypeStruct((N, D), jnp.float32),
    )(p, dg, x, wl, b.reshape(1, D), wr)


def kernel(x, edge_index, W1l, b1, W1r, W2l, b2, W2r):
    src = edge_index[0].reshape(NS, EPT)
    dst = edge_index[1].reshape(NS, EPT)
    pad = EPT_P - EPT
    srcp = jnp.pad(src, ((0, 0), (0, pad))).reshape(NS, NCH, CHUNK)
    dstp = jnp.pad(dst, ((0, 0), (0, pad)), constant_values=N).reshape(NS, NCH, CHUNK)

    sc_deg = _make_sc_pass(True)
    sc_plain = _make_sc_pass(False)

    xs = jnp.stack([x[:, :DH], x[:, DH:]])
    p1, dg = sc_deg(xs, srcp, dstp)
    h1 = _tc_pass(p1, dg, x, W1l, b1, W1r)
    hs = jnp.stack([h1[:, :DH], h1[:, DH:]])
    (p2,) = sc_plain(hs, srcp, dstp)
    h2 = _tc_pass(p2, dg, h1, W2l, b2, W2r)
    return h2
